# baseline (speedup 1.0000x reference)
import functools

import jax
import jax.numpy as jnp
from jax.experimental import pallas as pl
from jax.experimental.pallas import tpu as pltpu


def _dot1(a, b):
    return jnp.dot(a.astype(jnp.bfloat16), b.astype(jnp.bfloat16),
                   preferred_element_type=jnp.float32)


def _mm_kernel(x_ref, w_ref, o_ref):
    o_ref[...] = _dot1(x_ref[...], w_ref[...])


def _mm(x, wT):
    n, din = x.shape
    dout = wT.shape[1]
    return pl.pallas_call(
        _mm_kernel,
        out_shape=jax.ShapeDtypeStruct((n, dout), jnp.float32),
    )(x, wT)


def _adj_mm_prelu_kernel(adj_ref, x_ref, b_ref, a_ref, o_ref):
    o_ref[...] = jnp.zeros_like(o_ref)
    o_ref[...] += _dot1(adj_ref[...], x_ref[...])
    h = o_ref[...] + b_ref[...]
    a = a_ref[0]
    o_ref[...] = jnp.where(h >= 0, h, a * h)


def _adj_mm_prelu(adj, x, b, alpha, bm):
    n, k = adj.shape
    dout = x.shape[1]
    nm = n // bm
    return pl.pallas_call(
        _adj_mm_prelu_kernel,
        grid=(nm,),
        in_specs=[
            pl.BlockSpec((bm, k), lambda m: (m, 0)),
            pl.BlockSpec((k, dout), lambda m: (0, 0)),
            pl.BlockSpec((1, dout), lambda m: (0, 0)),
            pl.BlockSpec(memory_space=pltpu.SMEM),
        ],
        out_specs=pl.BlockSpec((bm, dout), lambda m: (m, 0)),
        out_shape=jax.ShapeDtypeStruct((n, dout), jnp.float32),
        compiler_params=pltpu.CompilerParams(
            dimension_semantics=("arbitrary",),
        ),
    )(adj, x, b.reshape(1, dout), alpha.reshape(1))


def _lin_sum_kernel(x_ref, w_ref, b_ref, h_ref, s_ref):
    m = pl.program_id(1)
    h = _dot1(x_ref[0], w_ref[0]) + b_ref[0]
    h_ref[0] = h

    @pl.when(m == 0)
    def _():
        s_ref[...] = jnp.zeros_like(s_ref)

    s_ref[0] += jnp.sum(h, axis=0, keepdims=True)


def _sqdev_kernel(h_ref, s_ref, v_ref, *, n):
    m = pl.program_id(1)
    d = h_ref[0] - s_ref[0] / n

    @pl.when(m == 0)
    def _():
        v_ref[...] = jnp.zeros_like(v_ref)

    v_ref[0] += jnp.sum(d * d, axis=0, keepdims=True)


def _bn_lin_sum_kernel(h_ref, s_ref, v_ref, g_ref, be_ref, w_ref, b_ref,
                       h2_ref, s2_ref, *, n):
    m = pl.program_id(1)
    mean = s_ref[0] / n
    scale = g_ref[0] / jnp.sqrt(v_ref[0] / n + 1e-5)
    xh = (h_ref[0] - mean) * scale + be_ref[0]
    xh = jnp.maximum(xh, 0.0)
    h2 = _dot1(xh, w_ref[0]) + b_ref[0]
    h2_ref[0] = h2

    @pl.when(m == 0)
    def _():
        s2_ref[...] = jnp.zeros_like(s2_ref)

    s2_ref[0] += jnp.sum(h2, axis=0, keepdims=True)


def _bn_apply_kernel(h_ref, s_ref, v_ref, g_ref, be_ref, o_ref, *, n):
    mean = s_ref[0] / n
    scale = g_ref[0] / jnp.sqrt(v_ref[0] / n + 1e-5)
    o_ref[0] = (h_ref[0] - mean) * scale + be_ref[0]


def _heads(x3, w1T, b1, g1, be1, w2T, b2, g2, be2, bm):
    t, n, din = x3.shape
    h = w1T.shape[2]
    out = w2T.shape[2]
    nm = n // bm
    const3 = lambda tt, m: (tt, 0, 0)
    row3 = lambda tt, m: (tt, m, 0)
    arb2 = pltpu.CompilerParams(dimension_semantics=("arbitrary", "arbitrary"))

    def lin_sum(x, wT, b, dout):
        return pl.pallas_call(
            _lin_sum_kernel,
            grid=(t, nm),
            in_specs=[
                pl.BlockSpec((1, bm, x.shape[2]), row3),
                pl.BlockSpec((1, x.shape[2], dout), const3),
                pl.BlockSpec((1, 1, dout), const3),
            ],
            out_specs=[
                pl.BlockSpec((1, bm, dout), row3),
                pl.BlockSpec((1, 1, dout), const3),
            ],
            out_shape=[
                jax.ShapeDtypeStruct((t, n, dout), jnp.float32),
                jax.ShapeDtypeStruct((t, 1, dout), jnp.float32),
            ],
            compiler_params=arb2,
        )(x, wT, b)

    def sqdev(hm, s, dout):
        return pl.pallas_call(
            functools.partial(_sqdev_kernel, n=n),
            grid=(t, nm),
            in_specs=[
                pl.BlockSpec((1, bm, dout), row3),
                pl.BlockSpec((1, 1, dout), const3),
            ],
            out_specs=pl.BlockSpec((1, 1, dout), const3),
            out_shape=jax.ShapeDtypeStruct((t, 1, dout), jnp.float32),
            compiler_params=arb2,
        )(hm, s)

    h1, s1 = lin_sum(x3, w1T, b1.reshape(t, 1, h), h)
    v1 = sqdev(h1, s1, h)

    h2, s2 = pl.pallas_call(
        functools.partial(_bn_lin_sum_kernel, n=n),
        grid=(t, nm),
        in_specs=[
            pl.BlockSpec((1, bm, h), row3),
            pl.BlockSpec((1, 1, h), const3),
            pl.BlockSpec((1, 1, h), const3),
            pl.BlockSpec((1, 1, h), const3),
            pl.BlockSpec((1, 1, h), const3),
            pl.BlockSpec((1, h, out), const3),
            pl.BlockSpec((1, 1, out), const3),
        ],
        out_specs=[
            pl.BlockSpec((1, bm, out), row3),
            pl.BlockSpec((1, 1, out), const3),
        ],
        out_shape=[
            jax.ShapeDtypeStruct((t, n, out), jnp.float32),
            jax.ShapeDtypeStruct((t, 1, out), jnp.float32),
        ],
        compiler_params=arb2,
    )(h1, s1, v1, g1.reshape(t, 1, h), be1.reshape(t, 1, h),
      w2T, b2.reshape(t, 1, out))
    v2 = sqdev(h2, s2, out)

    out3 = pl.pallas_call(
        functools.partial(_bn_apply_kernel, n=n),
        grid=(t, nm),
        in_specs=[
            pl.BlockSpec((1, bm, out), row3),
            pl.BlockSpec((1, 1, out), const3),
            pl.BlockSpec((1, 1, out), const3),
            pl.BlockSpec((1, 1, out), const3),
            pl.BlockSpec((1, 1, out), const3),
        ],
        out_specs=pl.BlockSpec((1, bm, out), row3),
        out_shape=jax.ShapeDtypeStruct((t, n, out), jnp.float32),
        compiler_params=arb2,
    )(h2, s2, v2, g2.reshape(t, 1, out), be2.reshape(t, 1, out))
    return out3


def _prompt_kernel(np_ref, ap_ref, fcnT_ref, fcaT_ref, prT_ref, pab_ref, pg_ref,
                   npr_ref, apr_ref, en_ref, ea_ref):
    npr = jnp.maximum(_dot1(np_ref[...], fcnT_ref[...]), 0.0)
    apr = jnp.maximum(_dot1(ap_ref[...], fcaT_ref[...]), 0.0)
    pab = pab_ref[...]
    pg = pg_ref[...]
    en = npr + jnp.maximum(_dot1(npr, prT_ref[...]) + pab, 0.0) + pg
    ea = apr + jnp.maximum(_dot1(apr, prT_ref[...]) + pab, 0.0) + pg
    npr_ref[...] = npr
    apr_ref[...] = apr
    en_ref[...] = en
    ea_ref[...] = ea


def _prompts(np_, ap_, fcnT, fcaT, prT, pab, pg):
    p, d2 = np_.shape
    sh = jax.ShapeDtypeStruct((p, d2), jnp.float32)
    return pl.pallas_call(
        _prompt_kernel,
        out_shape=[sh, sh, sh, sh],
    )(np_, ap_, fcnT, fcaT, prT, pab.reshape(1, d2), pg)


def kernel(feat, adj, ego_raw, nbr_raw, normal_prompt, abnormal_prompt, params):
    p = params
    n = adj.shape[0]
    bm = 400 if n % 400 == 0 else n
    bmh = 1000 if n % 1000 == 0 else n

    x1 = _mm(feat, p['gcn1_W'].T)
    emb = _adj_mm_prelu(adj, x1, p['gcn1_b'], p['gcn1_a'], bm)
    x2 = _mm(emb, p['gcn2_W'].T)
    z_pre = _adj_mm_prelu(adj, x2, p['gcn2_b'], p['gcn2_a'], bm)

    x3 = jnp.stack([z_pre, ego_raw, nbr_raw])
    w1T = jnp.stack([p['nc_W1'].T, p['ego_W1'].T, p['nbr_W1'].T])
    b1 = jnp.stack([p['nc_b1'], p['ego_b1'], p['nbr_b1']])
    g1 = jnp.stack([p['nc_g1'], p['ego_g1'], p['nbr_g1']])
    be1 = jnp.stack([p['nc_be1'], p['ego_be1'], p['nbr_be1']])
    w2T = jnp.stack([p['nc_W2'].T, p['ego_W2'].T, p['nbr_W2'].T])
    b2 = jnp.stack([p['nc_b2'], p['ego_b2'], p['nbr_b2']])
    g2 = jnp.stack([p['nc_g2'], p['ego_g2'], p['nbr_g2']])
    be2 = jnp.stack([p['nc_be2'], p['ego_be2'], p['nbr_be2']])
    out3 = _heads(x3, w1T, b1, g1, be1, w2T, b2, g2, be2, bmh)
    z = out3[0]
    h_ego = out3[1]
    h_nbr = out3[2]

    npr, apr, en, ea = _prompts(
        normal_prompt, abnormal_prompt,
        p['fcn_W'].T, p['fca_W'].T, p['pr_aW'].T, p['pr_ab'], p['pr_glob'])

    return (h_ego, h_nbr, npr, apr, en, ea, z)


# speedup vs baseline: 1.1278x; 1.1278x over previous
import functools

import jax
import jax.numpy as jnp
from jax.experimental import pallas as pl
from jax.experimental.pallas import tpu as pltpu


def _dot1(a, b):
    return jnp.dot(a.astype(jnp.bfloat16), b.astype(jnp.bfloat16),
                   preferred_element_type=jnp.float32)


def _gcn_layer_kernel(x_in_ref, w_ref, adj_ref, b_ref, a_ref, o_ref, xv_ref):
    @pl.when(pl.program_id(0) == 0)
    def _():
        xv_ref[...] = _dot1(x_in_ref[...], w_ref[...])

    o_ref[...] = jnp.zeros_like(o_ref)
    o_ref[...] += _dot1(adj_ref[...], xv_ref[...])
    h = o_ref[...] + b_ref[...]
    a = a_ref[0]
    o_ref[...] = jnp.where(h >= 0, h, a * h)


def _gcn_layer(x_in, wT, adj, b, alpha, bm):
    n, k = adj.shape
    dout = wT.shape[1]
    return pl.pallas_call(
        _gcn_layer_kernel,
        grid=(n // bm,),
        in_specs=[
            pl.BlockSpec((n, wT.shape[0]), lambda m: (0, 0)),
            pl.BlockSpec((wT.shape[0], dout), lambda m: (0, 0)),
            pl.BlockSpec((bm, k), lambda m: (m, 0)),
            pl.BlockSpec((1, dout), lambda m: (0, 0)),
            pl.BlockSpec(memory_space=pltpu.SMEM),
        ],
        out_specs=pl.BlockSpec((bm, dout), lambda m: (m, 0)),
        out_shape=jax.ShapeDtypeStruct((n, dout), jnp.float32),
        scratch_shapes=[pltpu.VMEM((n, dout), jnp.float32)],
        compiler_params=pltpu.CompilerParams(
            dimension_semantics=("arbitrary",),
        ),
    )(x_in, wT, adj, b.reshape(1, dout), alpha.reshape(1))


def _lin_stats_kernel(x_ref, w_ref, b_ref, h_ref, s_ref, c_ref, ssc_ref):
    m = pl.program_id(1)
    h = _dot1(x_ref[0], w_ref[0]) + b_ref[0]
    h_ref[0] = h

    @pl.when(m == 0)
    def _():
        c_ref[0] = jnp.mean(h, axis=0, keepdims=True)
        s_ref[...] = jnp.zeros_like(s_ref)
        ssc_ref[...] = jnp.zeros_like(ssc_ref)

    d = h - c_ref[0]
    s_ref[0] += jnp.sum(h, axis=0, keepdims=True)
    ssc_ref[0] += jnp.sum(d * d, axis=0, keepdims=True)


def _bn_of(s, c, ssc, g, be, n):
    mean = s / n
    var = ssc / n - (mean - c) * (mean - c)
    scale = g / jnp.sqrt(var + 1e-5)
    return mean, scale


def _bn_lin_stats_kernel(h_ref, s_ref, c_ref, ssc_ref, g_ref, be_ref,
                         w_ref, b_ref, h2_ref, s2_ref, c2_ref, ssc2_ref, *, n):
    m = pl.program_id(1)
    mean, scale = _bn_of(s_ref[0], c_ref[0], ssc_ref[0], g_ref[0], be_ref[0], n)
    xh = (h_ref[0] - mean) * scale + be_ref[0]
    xh = jnp.maximum(xh, 0.0)
    h2 = _dot1(xh, w_ref[0]) + b_ref[0]
    h2_ref[0] = h2

    @pl.when(m == 0)
    def _():
        c2_ref[0] = jnp.mean(h2, axis=0, keepdims=True)
        s2_ref[...] = jnp.zeros_like(s2_ref)
        ssc2_ref[...] = jnp.zeros_like(ssc2_ref)

    d = h2 - c2_ref[0]
    s2_ref[0] += jnp.sum(h2, axis=0, keepdims=True)
    ssc2_ref[0] += jnp.sum(d * d, axis=0, keepdims=True)


def _bn_apply_kernel(h_ref, s_ref, c_ref, ssc_ref, g_ref, be_ref, o_ref, *, n):
    mean, scale = _bn_of(s_ref[0], c_ref[0], ssc_ref[0], g_ref[0], be_ref[0], n)
    o_ref[0] = (h_ref[0] - mean) * scale + be_ref[0]


def _heads(x3, w1T, b1, g1, be1, w2T, b2, g2, be2, bm):
    t, n, din = x3.shape
    h = w1T.shape[2]
    out = w2T.shape[2]
    nm = n // bm
    const3 = lambda tt, m: (tt, 0, 0)
    row3 = lambda tt, m: (tt, m, 0)
    arb2 = pltpu.CompilerParams(dimension_semantics=("arbitrary", "arbitrary"))

    def stat_spec(d):
        return pl.BlockSpec((1, 1, d), const3)

    def stat_shape(d):
        return jax.ShapeDtypeStruct((t, 1, d), jnp.float32)

    h1, s1, c1, ssc1 = pl.pallas_call(
        _lin_stats_kernel,
        grid=(t, nm),
        in_specs=[
            pl.BlockSpec((1, bm, din), row3),
            pl.BlockSpec((1, din, h), const3),
            stat_spec(h),
        ],
        out_specs=[pl.BlockSpec((1, bm, h), row3),
                   stat_spec(h), stat_spec(h), stat_spec(h)],
        out_shape=[jax.ShapeDtypeStruct((t, n, h), jnp.float32),
                   stat_shape(h), stat_shape(h), stat_shape(h)],
        compiler_params=arb2,
    )(x3, w1T, b1.reshape(t, 1, h))

    h2, s2, c2, ssc2 = pl.pallas_call(
        functools.partial(_bn_lin_stats_kernel, n=n),
        grid=(t, nm),
        in_specs=[
            pl.BlockSpec((1, bm, h), row3),
            stat_spec(h), stat_spec(h), stat_spec(h),
            stat_spec(h), stat_spec(h),
            pl.BlockSpec((1, h, out), const3),
            stat_spec(out),
        ],
        out_specs=[pl.BlockSpec((1, bm, out), row3),
                   stat_spec(out), stat_spec(out), stat_spec(out)],
        out_shape=[jax.ShapeDtypeStruct((t, n, out), jnp.float32),
                   stat_shape(out), stat_shape(out), stat_shape(out)],
        compiler_params=arb2,
    )(h1, s1, c1, ssc1, g1.reshape(t, 1, h), be1.reshape(t, 1, h),
      w2T, b2.reshape(t, 1, out))

    out3 = pl.pallas_call(
        functools.partial(_bn_apply_kernel, n=n),
        grid=(t, nm),
        in_specs=[
            pl.BlockSpec((1, bm, out), row3),
            stat_spec(out), stat_spec(out), stat_spec(out),
            stat_spec(out), stat_spec(out),
        ],
        out_specs=pl.BlockSpec((1, bm, out), row3),
        out_shape=jax.ShapeDtypeStruct((t, n, out), jnp.float32),
        compiler_params=arb2,
    )(h2, s2, c2, ssc2, g2.reshape(t, 1, out), be2.reshape(t, 1, out))
    return out3


def _prompt_kernel(np_ref, ap_ref, fcnT_ref, fcaT_ref, prT_ref, pab_ref, pg_ref,
                   npr_ref, apr_ref, en_ref, ea_ref):
    npr = jnp.maximum(_dot1(np_ref[...], fcnT_ref[...]), 0.0)
    apr = jnp.maximum(_dot1(ap_ref[...], fcaT_ref[...]), 0.0)
    pab = pab_ref[...]
    pg = pg_ref[...]
    en = npr + jnp.maximum(_dot1(npr, prT_ref[...]) + pab, 0.0) + pg
    ea = apr + jnp.maximum(_dot1(apr, prT_ref[...]) + pab, 0.0) + pg
    npr_ref[...] = npr
    apr_ref[...] = apr
    en_ref[...] = en
    ea_ref[...] = ea


def _prompts(np_, ap_, fcnT, fcaT, prT, pab, pg):
    p, d2 = np_.shape
    sh = jax.ShapeDtypeStruct((p, d2), jnp.float32)
    return pl.pallas_call(
        _prompt_kernel,
        out_shape=[sh, sh, sh, sh],
    )(np_, ap_, fcnT, fcaT, prT, pab.reshape(1, d2), pg)


def kernel(feat, adj, ego_raw, nbr_raw, normal_prompt, abnormal_prompt, params):
    p = params
    n = adj.shape[0]
    bm = 400 if n % 400 == 0 else n
    bmh = 1000 if n % 1000 == 0 else n

    emb = _gcn_layer(feat, p['gcn1_W'].T, adj, p['gcn1_b'], p['gcn1_a'], bm)
    z_pre = _gcn_layer(emb, p['gcn2_W'].T, adj, p['gcn2_b'], p['gcn2_a'], bm)

    x3 = jnp.stack([z_pre, ego_raw, nbr_raw])
    w1T = jnp.stack([p['nc_W1'].T, p['ego_W1'].T, p['nbr_W1'].T])
    b1 = jnp.stack([p['nc_b1'], p['ego_b1'], p['nbr_b1']])
    g1 = jnp.stack([p['nc_g1'], p['ego_g1'], p['nbr_g1']])
    be1 = jnp.stack([p['nc_be1'], p['ego_be1'], p['nbr_be1']])
    w2T = jnp.stack([p['nc_W2'].T, p['ego_W2'].T, p['nbr_W2'].T])
    b2 = jnp.stack([p['nc_b2'], p['ego_b2'], p['nbr_b2']])
    g2 = jnp.stack([p['nc_g2'], p['ego_g2'], p['nbr_g2']])
    be2 = jnp.stack([p['nc_be2'], p['ego_be2'], p['nbr_be2']])
    out3 = _heads(x3, w1T, b1, g1, be1, w2T, b2, g2, be2, bmh)
    z = out3[0]
    h_ego = out3[1]
    h_nbr = out3[2]

    npr, apr, en, ea = _prompts(
        normal_prompt, abnormal_prompt,
        p['fcn_W'].T, p['fca_W'].T, p['pr_aW'].T, p['pr_ab'], p['pr_glob'])

    return (h_ego, h_nbr, npr, apr, en, ea, z)


# speedup vs baseline: 1.2968x; 1.1498x over previous
import functools

import jax
import jax.numpy as jnp
from jax.experimental import pallas as pl
from jax.experimental.pallas import tpu as pltpu


def _dot1(a, b):
    return jnp.dot(a.astype(jnp.bfloat16), b.astype(jnp.bfloat16),
                   preferred_element_type=jnp.float32)


def _gcn_layer_kernel(x_in_ref, w_ref, adj_ref, b_ref, a_ref, o_ref, xv_ref):
    @pl.when(pl.program_id(0) == 0)
    def _():
        xv_ref[...] = _dot1(x_in_ref[...], w_ref[...])

    o_ref[...] = jnp.zeros_like(o_ref)
    o_ref[...] += _dot1(adj_ref[...], xv_ref[...])
    h = o_ref[...] + b_ref[...]
    a = a_ref[0]
    o_ref[...] = jnp.where(h >= 0, h, a * h)


def _gcn_layer(x_in, wT, adj, b, alpha, bm):
    n, k = adj.shape
    dout = wT.shape[1]
    return pl.pallas_call(
        _gcn_layer_kernel,
        grid=(n // bm,),
        in_specs=[
            pl.BlockSpec((n, wT.shape[0]), lambda m: (0, 0)),
            pl.BlockSpec((wT.shape[0], dout), lambda m: (0, 0)),
            pl.BlockSpec((bm, k), lambda m: (m, 0)),
            pl.BlockSpec((1, dout), lambda m: (0, 0)),
            pl.BlockSpec(memory_space=pltpu.SMEM),
        ],
        out_specs=pl.BlockSpec((bm, dout), lambda m: (m, 0)),
        out_shape=jax.ShapeDtypeStruct((n, dout), jnp.float32),
        scratch_shapes=[pltpu.VMEM((n, dout), jnp.float32)],
        compiler_params=pltpu.CompilerParams(
            dimension_semantics=("arbitrary",),
        ),
    )(x_in, wT, adj, b.reshape(1, dout), alpha.reshape(1))


def _lin_stats_kernel(xz_ref, xe_ref, xn_ref, w_ref, b_ref,
                      hz_ref, he_ref, hn_ref, s_ref, c_ref, ssc_ref):
    m = pl.program_id(0)
    for i, (x_ref, h_ref) in enumerate(
            ((xz_ref, hz_ref), (xe_ref, he_ref), (xn_ref, hn_ref))):
        h = _dot1(x_ref[...], w_ref[i]) + b_ref[i]
        h_ref[...] = h

        @pl.when(m == 0)
        def _(h=h, i=i):
            c_ref[i] = jnp.mean(h, axis=0, keepdims=True)
            s_ref[i] = jnp.zeros_like(s_ref[i])
            ssc_ref[i] = jnp.zeros_like(ssc_ref[i])

        d = h - c_ref[i]
        s_ref[i] += jnp.sum(h, axis=0, keepdims=True)
        ssc_ref[i] += jnp.sum(d * d, axis=0, keepdims=True)


def _bn_of(s, c, ssc, g, n):
    mean = s / n
    var = ssc / n - (mean - c) * (mean - c)
    scale = g / jnp.sqrt(var + 1e-5)
    return mean, scale


def _bn_lin_stats_kernel(hz_ref, he_ref, hn_ref, s_ref, c_ref, ssc_ref,
                         g_ref, be_ref, w_ref, b_ref,
                         oz_ref, oe_ref, on_ref, s2_ref, c2_ref, ssc2_ref, *, n):
    m = pl.program_id(0)
    for i, (h_ref, o_ref) in enumerate(
            ((hz_ref, oz_ref), (he_ref, oe_ref), (hn_ref, on_ref))):
        mean, scale = _bn_of(s_ref[i], c_ref[i], ssc_ref[i], g_ref[i], n)
        xh = (h_ref[...] - mean) * scale + be_ref[i]
        xh = jnp.maximum(xh, 0.0)
        h2 = _dot1(xh, w_ref[i]) + b_ref[i]
        o_ref[...] = h2

        @pl.when(m == 0)
        def _(h2=h2, i=i):
            c2_ref[i] = jnp.mean(h2, axis=0, keepdims=True)
            s2_ref[i] = jnp.zeros_like(s2_ref[i])
            ssc2_ref[i] = jnp.zeros_like(ssc2_ref[i])

        d = h2 - c2_ref[i]
        s2_ref[i] += jnp.sum(h2, axis=0, keepdims=True)
        ssc2_ref[i] += jnp.sum(d * d, axis=0, keepdims=True)


def _bn_apply_prompt_kernel(hz_ref, he_ref, hn_ref, s_ref, c_ref, ssc_ref,
                            g_ref, be_ref,
                            np_ref, ap_ref, fcnT_ref, fcaT_ref, prT_ref,
                            pab_ref, pg_ref,
                            oz_ref, oe_ref, on_ref,
                            npr_ref, apr_ref, en_ref, ea_ref, *, n):
    for i, (h_ref, o_ref) in enumerate(
            ((hz_ref, oz_ref), (he_ref, oe_ref), (hn_ref, on_ref))):
        mean, scale = _bn_of(s_ref[i], c_ref[i], ssc_ref[i], g_ref[i], n)
        o_ref[...] = (h_ref[...] - mean) * scale + be_ref[i]

    @pl.when(pl.program_id(0) == 0)
    def _():
        npr = jnp.maximum(_dot1(np_ref[...], fcnT_ref[...]), 0.0)
        apr = jnp.maximum(_dot1(ap_ref[...], fcaT_ref[...]), 0.0)
        pab = pab_ref[...]
        pg = pg_ref[...]
        npr_ref[...] = npr
        apr_ref[...] = apr
        en_ref[...] = npr + jnp.maximum(_dot1(npr, prT_ref[...]) + pab, 0.0) + pg
        ea_ref[...] = apr + jnp.maximum(_dot1(apr, prT_ref[...]) + pab, 0.0) + pg


def _heads_and_prompts(z_pre, ego, nbr, prompts_in, p, bm):
    n, din = z_pre.shape
    w1T = jnp.stack([p['nc_W1'].T, p['ego_W1'].T, p['nbr_W1'].T])
    b1 = jnp.stack([p['nc_b1'], p['ego_b1'], p['nbr_b1']]).reshape(3, 1, -1)
    g1 = jnp.stack([p['nc_g1'], p['ego_g1'], p['nbr_g1']]).reshape(3, 1, -1)
    be1 = jnp.stack([p['nc_be1'], p['ego_be1'], p['nbr_be1']]).reshape(3, 1, -1)
    w2T = jnp.stack([p['nc_W2'].T, p['ego_W2'].T, p['nbr_W2'].T])
    b2 = jnp.stack([p['nc_b2'], p['ego_b2'], p['nbr_b2']]).reshape(3, 1, -1)
    g2 = jnp.stack([p['nc_g2'], p['ego_g2'], p['nbr_g2']]).reshape(3, 1, -1)
    be2 = jnp.stack([p['nc_be2'], p['ego_be2'], p['nbr_be2']]).reshape(3, 1, -1)
    h = w1T.shape[2]
    out = w2T.shape[2]
    nm = n // bm
    row = lambda m: (m, 0)
    const2 = lambda m: (0, 0)
    const3 = lambda m: (0, 0, 0)
    arb = pltpu.CompilerParams(dimension_semantics=("arbitrary",))

    def rowspec(d):
        return pl.BlockSpec((bm, d), row)

    def statspec(d):
        return pl.BlockSpec((3, 1, d), const3)

    def statshape(d):
        return jax.ShapeDtypeStruct((3, 1, d), jnp.float32)

    def wspec(a, b):
        return pl.BlockSpec((3, a, b), const3)

    hz, he, hn, s1, c1, ssc1 = pl.pallas_call(
        _lin_stats_kernel,
        grid=(nm,),
        in_specs=[rowspec(din), rowspec(din), rowspec(din),
                  wspec(din, h), statspec(h)],
        out_specs=[rowspec(h), rowspec(h), rowspec(h),
                   statspec(h), statspec(h), statspec(h)],
        out_shape=[jax.ShapeDtypeStruct((n, h), jnp.float32)] * 3 +
                  [statshape(h)] * 3,
        compiler_params=arb,
    )(z_pre, ego, nbr, w1T, b1)

    h2z, h2e, h2n, s2, c2, ssc2 = pl.pallas_call(
        functools.partial(_bn_lin_stats_kernel, n=n),
        grid=(nm,),
        in_specs=[rowspec(h), rowspec(h), rowspec(h),
                  statspec(h), statspec(h), statspec(h),
                  statspec(h), statspec(h),
                  wspec(h, out), statspec(out)],
        out_specs=[rowspec(out), rowspec(out), rowspec(out),
                   statspec(out), statspec(out), statspec(out)],
        out_shape=[jax.ShapeDtypeStruct((n, out), jnp.float32)] * 3 +
                  [statshape(out)] * 3,
        compiler_params=arb,
    )(hz, he, hn, s1, c1, ssc1, g1, be1, w2T, b2)

    np_, ap_, fcnT, fcaT, prT, pab, pg = prompts_in
    d2 = np_.shape[1]
    psh = jax.ShapeDtypeStruct((1, d2), jnp.float32)
    pspec = pl.BlockSpec((1, d2), const2)
    wspec2 = pl.BlockSpec((d2, d2), const2)
    z, oe, on, npr, apr, en, ea = pl.pallas_call(
        functools.partial(_bn_apply_prompt_kernel, n=n),
        grid=(nm,),
        in_specs=[rowspec(out), rowspec(out), rowspec(out),
                  statspec(out), statspec(out), statspec(out),
                  statspec(out), statspec(out),
                  pspec, pspec, wspec2, wspec2, wspec2, pspec, pspec],
        out_specs=[rowspec(out), rowspec(out), rowspec(out),
                   pspec, pspec, pspec, pspec],
        out_shape=[jax.ShapeDtypeStruct((n, out), jnp.float32)] * 3 +
                  [psh] * 4,
        compiler_params=arb,
    )(h2z, h2e, h2n, s2, c2, ssc2, g2, be2,
      np_, ap_, fcnT, fcaT, prT, pab, pg)
    return z, oe, on, npr, apr, en, ea


def kernel(feat, adj, ego_raw, nbr_raw, normal_prompt, abnormal_prompt, params):
    p = params
    n = adj.shape[0]
    bm = 400 if n % 400 == 0 else n
    bmh = 1000 if n % 1000 == 0 else n

    emb = _gcn_layer(feat, p['gcn1_W'].T, adj, p['gcn1_b'], p['gcn1_a'], bm)
    z_pre = _gcn_layer(emb, p['gcn2_W'].T, adj, p['gcn2_b'], p['gcn2_a'], bm)

    prompts_in = (normal_prompt, abnormal_prompt,
                  p['fcn_W'].T, p['fca_W'].T, p['pr_aW'].T,
                  p['pr_ab'].reshape(1, -1), p['pr_glob'])
    z, h_ego, h_nbr, npr, apr, en, ea = _heads_and_prompts(
        z_pre, ego_raw, nbr_raw, prompts_in, p, bmh)

    return (h_ego, h_nbr, npr, apr, en, ea, z)


# speedup vs baseline: 1.3344x; 1.0290x over previous
import functools

import jax
import jax.numpy as jnp
from jax.experimental import pallas as pl
from jax.experimental.pallas import tpu as pltpu


def _dot1(a, b):
    return jnp.dot(a.astype(jnp.bfloat16), b.astype(jnp.bfloat16),
                   preferred_element_type=jnp.float32)


def _head1_block(h, i, m, s_ref, c_ref, ssc_ref):
    @pl.when(m == 0)
    def _():
        c_ref[i] = jnp.mean(h, axis=0, keepdims=True)
        s_ref[i] = jnp.zeros_like(s_ref[i])
        ssc_ref[i] = jnp.zeros_like(ssc_ref[i])

    d = h - c_ref[i]
    s_ref[i] += jnp.sum(h, axis=0, keepdims=True)
    ssc_ref[i] += jnp.sum(d * d, axis=0, keepdims=True)


def _gcn1_kernel(x_in_ref, w_ref, adj_ref, b_ref, a_ref, xe_ref, xn_ref,
                 hw_ref, hb_ref,
                 o_ref, he_ref, hn_ref, s_ref, c_ref, ssc_ref, xv_ref):
    m = pl.program_id(0)

    @pl.when(m == 0)
    def _():
        xv_ref[...] = _dot1(x_in_ref[...], w_ref[...])

    o_ref[...] = jnp.zeros_like(o_ref)
    o_ref[...] += _dot1(adj_ref[...], xv_ref[...])
    h = o_ref[...] + b_ref[...]
    a = a_ref[0]
    o_ref[...] = jnp.where(h >= 0, h, a * h)

    for i, (x_ref, h_ref) in enumerate(((xe_ref, he_ref), (xn_ref, hn_ref))):
        h1 = _dot1(x_ref[...], hw_ref[i]) + hb_ref[i]
        h_ref[...] = h1
        _head1_block(h1, i, m, s_ref, c_ref, ssc_ref)


def _gcn1_layer(x_in, wT, adj, b, alpha, ego, nbr, hw, hb, bm):
    n, k = adj.shape
    dout = wT.shape[1]
    hh = hw.shape[2]
    row = lambda m: (m, 0)
    const2 = lambda m: (0, 0)
    const3 = lambda m: (0, 0, 0)
    return pl.pallas_call(
        _gcn1_kernel,
        grid=(n // bm,),
        in_specs=[
            pl.BlockSpec((n, wT.shape[0]), const2),
            pl.BlockSpec((wT.shape[0], dout), const2),
            pl.BlockSpec((bm, k), row),
            pl.BlockSpec((1, dout), const2),
            pl.BlockSpec(memory_space=pltpu.SMEM),
            pl.BlockSpec((bm, ego.shape[1]), row),
            pl.BlockSpec((bm, nbr.shape[1]), row),
            pl.BlockSpec((2, ego.shape[1], hh), const3),
            pl.BlockSpec((2, 1, hh), const3),
        ],
        out_specs=[
            pl.BlockSpec((bm, dout), row),
            pl.BlockSpec((bm, hh), row),
            pl.BlockSpec((bm, hh), row),
            pl.BlockSpec((2, 1, hh), const3),
            pl.BlockSpec((2, 1, hh), const3),
            pl.BlockSpec((2, 1, hh), const3),
        ],
        out_shape=[
            jax.ShapeDtypeStruct((n, dout), jnp.float32),
            jax.ShapeDtypeStruct((n, hh), jnp.float32),
            jax.ShapeDtypeStruct((n, hh), jnp.float32),
            jax.ShapeDtypeStruct((2, 1, hh), jnp.float32),
            jax.ShapeDtypeStruct((2, 1, hh), jnp.float32),
            jax.ShapeDtypeStruct((2, 1, hh), jnp.float32),
        ],
        scratch_shapes=[pltpu.VMEM((n, dout), jnp.float32)],
        compiler_params=pltpu.CompilerParams(
            dimension_semantics=("arbitrary",),
        ),
    )(x_in, wT, adj, b.reshape(1, dout), alpha.reshape(1), ego, nbr, hw, hb)


def _gcn2_kernel(x_in_ref, w_ref, adj_ref, b_ref, a_ref, hw_ref, hb_ref,
                 o_ref, hz_ref, s_ref, c_ref, ssc_ref, xv_ref):
    m = pl.program_id(0)

    @pl.when(m == 0)
    def _():
        xv_ref[...] = _dot1(x_in_ref[...], w_ref[...])

    o_ref[...] = jnp.zeros_like(o_ref)
    o_ref[...] += _dot1(adj_ref[...], xv_ref[...])
    h = o_ref[...] + b_ref[...]
    a = a_ref[0]
    zb = jnp.where(h >= 0, h, a * h)
    o_ref[...] = zb

    h1 = _dot1(zb, hw_ref[0]) + hb_ref[0]
    hz_ref[...] = h1
    _head1_block(h1, 0, m, s_ref, c_ref, ssc_ref)


def _gcn2_layer(x_in, wT, adj, b, alpha, hw, hb, bm):
    n, k = adj.shape
    dout = wT.shape[1]
    hh = hw.shape[2]
    row = lambda m: (m, 0)
    const2 = lambda m: (0, 0)
    const3 = lambda m: (0, 0, 0)
    return pl.pallas_call(
        _gcn2_kernel,
        grid=(n // bm,),
        in_specs=[
            pl.BlockSpec((n, wT.shape[0]), const2),
            pl.BlockSpec((wT.shape[0], dout), const2),
            pl.BlockSpec((bm, k), row),
            pl.BlockSpec((1, dout), const2),
            pl.BlockSpec(memory_space=pltpu.SMEM),
            pl.BlockSpec((1, dout, hh), const3),
            pl.BlockSpec((1, 1, hh), const3),
        ],
        out_specs=[
            pl.BlockSpec((bm, dout), row),
            pl.BlockSpec((bm, hh), row),
            pl.BlockSpec((1, 1, hh), const3),
            pl.BlockSpec((1, 1, hh), const3),
            pl.BlockSpec((1, 1, hh), const3),
        ],
        out_shape=[
            jax.ShapeDtypeStruct((n, dout), jnp.float32),
            jax.ShapeDtypeStruct((n, hh), jnp.float32),
            jax.ShapeDtypeStruct((1, 1, hh), jnp.float32),
            jax.ShapeDtypeStruct((1, 1, hh), jnp.float32),
            jax.ShapeDtypeStruct((1, 1, hh), jnp.float32),
        ],
        scratch_shapes=[pltpu.VMEM((n, dout), jnp.float32)],
        compiler_params=pltpu.CompilerParams(
            dimension_semantics=("arbitrary",),
        ),
    )(x_in, wT, adj, b.reshape(1, dout), alpha.reshape(1), hw, hb)


def _bn_of(s, c, ssc, g, n):
    mean = s / n
    var = ssc / n - (mean - c) * (mean - c)
    scale = g / jnp.sqrt(var + 1e-5)
    return mean, scale


def _bn_lin_stats_kernel(hz_ref, he_ref, hn_ref, sz_ref, cz_ref, sscz_ref,
                         se_ref, ce_ref, ssce_ref,
                         g_ref, be_ref, w_ref, b_ref,
                         oz_ref, oe_ref, on_ref, s2_ref, c2_ref, ssc2_ref, *, n):
    m = pl.program_id(0)
    stats = ((sz_ref[0], cz_ref[0], sscz_ref[0]),
             (se_ref[0], ce_ref[0], ssce_ref[0]),
             (se_ref[1], ce_ref[1], ssce_ref[1]))
    for i, (h_ref, o_ref) in enumerate(
            ((hz_ref, oz_ref), (he_ref, oe_ref), (hn_ref, on_ref))):
        s_i, c_i, ssc_i = stats[i]
        mean, scale = _bn_of(s_i, c_i, ssc_i, g_ref[i], n)
        xh = (h_ref[...] - mean) * scale + be_ref[i]
        xh = jnp.maximum(xh, 0.0)
        h2 = _dot1(xh, w_ref[i]) + b_ref[i]
        o_ref[...] = h2

        @pl.when(m == 0)
        def _(h2=h2, i=i):
            c2_ref[i] = jnp.mean(h2, axis=0, keepdims=True)
            s2_ref[i] = jnp.zeros_like(s2_ref[i])
            ssc2_ref[i] = jnp.zeros_like(ssc2_ref[i])

        d = h2 - c2_ref[i]
        s2_ref[i] += jnp.sum(h2, axis=0, keepdims=True)
        ssc2_ref[i] += jnp.sum(d * d, axis=0, keepdims=True)


def _bn_apply_prompt_kernel(hz_ref, he_ref, hn_ref, s_ref, c_ref, ssc_ref,
                            g_ref, be_ref,
                            np_ref, ap_ref, fcnT_ref, fcaT_ref, prT_ref,
                            pab_ref, pg_ref,
                            oz_ref, oe_ref, on_ref,
                            npr_ref, apr_ref, en_ref, ea_ref, *, n):
    for i, (h_ref, o_ref) in enumerate(
            ((hz_ref, oz_ref), (he_ref, oe_ref), (hn_ref, on_ref))):
        mean, scale = _bn_of(s_ref[i], c_ref[i], ssc_ref[i], g_ref[i], n)
        o_ref[...] = (h_ref[...] - mean) * scale + be_ref[i]

    @pl.when(pl.program_id(0) == 0)
    def _():
        npr = jnp.maximum(_dot1(np_ref[...], fcnT_ref[...]), 0.0)
        apr = jnp.maximum(_dot1(ap_ref[...], fcaT_ref[...]), 0.0)
        pab = pab_ref[...]
        pg = pg_ref[...]
        npr_ref[...] = npr
        apr_ref[...] = apr
        en_ref[...] = npr + jnp.maximum(_dot1(npr, prT_ref[...]) + pab, 0.0) + pg
        ea_ref[...] = apr + jnp.maximum(_dot1(apr, prT_ref[...]) + pab, 0.0) + pg


def _heads_and_prompts(h1z, h1e, h1n, stats_z, stats_en, prompts_in, p, bm):
    n, h = h1z.shape
    g1 = jnp.stack([p['nc_g1'], p['ego_g1'], p['nbr_g1']]).reshape(3, 1, -1)
    be1 = jnp.stack([p['nc_be1'], p['ego_be1'], p['nbr_be1']]).reshape(3, 1, -1)
    w2T = jnp.stack([p['nc_W2'].T, p['ego_W2'].T, p['nbr_W2'].T])
    b2 = jnp.stack([p['nc_b2'], p['ego_b2'], p['nbr_b2']]).reshape(3, 1, -1)
    g2 = jnp.stack([p['nc_g2'], p['ego_g2'], p['nbr_g2']]).reshape(3, 1, -1)
    be2 = jnp.stack([p['nc_be2'], p['ego_be2'], p['nbr_be2']]).reshape(3, 1, -1)
    out = w2T.shape[2]
    nm = n // bm
    row = lambda m: (m, 0)
    const2 = lambda m: (0, 0)
    const3 = lambda m: (0, 0, 0)
    arb = pltpu.CompilerParams(dimension_semantics=("arbitrary",))

    def rowspec(d):
        return pl.BlockSpec((bm, d), row)

    def statspec(d, t=3):
        return pl.BlockSpec((t, 1, d), const3)

    def statshape(d):
        return jax.ShapeDtypeStruct((3, 1, d), jnp.float32)

    def wspec(a, b):
        return pl.BlockSpec((3, a, b), const3)

    sz, cz, sscz = stats_z
    sen, cen, sscen = stats_en
    h2z, h2e, h2n, s2, c2, ssc2 = pl.pallas_call(
        functools.partial(_bn_lin_stats_kernel, n=n),
        grid=(nm,),
        in_specs=[rowspec(h), rowspec(h), rowspec(h),
                  statspec(h, 1), statspec(h, 1), statspec(h, 1),
                  statspec(h, 2), statspec(h, 2), statspec(h, 2),
                  statspec(h), statspec(h),
                  wspec(h, out), statspec(out)],
        out_specs=[rowspec(out), rowspec(out), rowspec(out),
                   statspec(out), statspec(out), statspec(out)],
        out_shape=[jax.ShapeDtypeStruct((n, out), jnp.float32)] * 3 +
                  [statshape(out)] * 3,
        compiler_params=arb,
    )(h1z, h1e, h1n, sz, cz, sscz, sen, cen, sscen, g1, be1, w2T, b2)

    np_, ap_, fcnT, fcaT, prT, pab, pg = prompts_in
    d2 = np_.shape[1]
    psh = jax.ShapeDtypeStruct((1, d2), jnp.float32)
    pspec = pl.BlockSpec((1, d2), const2)
    wspec2 = pl.BlockSpec((d2, d2), const2)
    z, oe, on, npr, apr, en, ea = pl.pallas_call(
        functools.partial(_bn_apply_prompt_kernel, n=n),
        grid=(nm,),
        in_specs=[rowspec(out), rowspec(out), rowspec(out),
                  statspec(out), statspec(out), statspec(out),
                  statspec(out), statspec(out),
                  pspec, pspec, wspec2, wspec2, wspec2, pspec, pspec],
        out_specs=[rowspec(out), rowspec(out), rowspec(out),
                   pspec, pspec, pspec, pspec],
        out_shape=[jax.ShapeDtypeStruct((n, out), jnp.float32)] * 3 +
                  [psh] * 4,
        compiler_params=arb,
    )(h2z, h2e, h2n, s2, c2, ssc2, g2, be2,
      np_, ap_, fcnT, fcaT, prT, pab, pg)
    return z, oe, on, npr, apr, en, ea


def kernel(feat, adj, ego_raw, nbr_raw, normal_prompt, abnormal_prompt, params):
    p = params
    n = adj.shape[0]
    bm = 400 if n % 400 == 0 else n
    bmh = 1000 if n % 1000 == 0 else n

    hw_en = jnp.stack([p['ego_W1'].T, p['nbr_W1'].T])
    hb_en = jnp.stack([p['ego_b1'], p['nbr_b1']]).reshape(2, 1, -1)
    emb, h1e, h1n, s_en, c_en, ssc_en = _gcn1_layer(
        feat, p['gcn1_W'].T, adj, p['gcn1_b'], p['gcn1_a'],
        ego_raw, nbr_raw, hw_en, hb_en, bm)

    hw_nc = p['nc_W1'].T.reshape(1, *p['nc_W1'].shape)
    hb_nc = p['nc_b1'].reshape(1, 1, -1)
    z_pre, h1z, s_z, c_z, ssc_z = _gcn2_layer(
        emb, p['gcn2_W'].T, adj, p['gcn2_b'], p['gcn2_a'], hw_nc, hb_nc, bm)
    del z_pre

    prompts_in = (normal_prompt, abnormal_prompt,
                  p['fcn_W'].T, p['fca_W'].T, p['pr_aW'].T,
                  p['pr_ab'].reshape(1, -1), p['pr_glob'])
    z, h_ego, h_nbr, npr, apr, en, ea = _heads_and_prompts(
        h1z, h1e, h1n, (s_z, c_z, ssc_z), (s_en, c_en, ssc_en),
        prompts_in, p, bmh)

    return (h_ego, h_nbr, npr, apr, en, ea, z)
